# Initial kernel scaffold; baseline (speedup 1.0000x reference)
import jax
import jax.numpy as jnp
from jax.experimental import pallas as pl


def kernel(X_train, X_test, y_train):
    raise NotImplementedError("write your pallas kernel here")


# speedup vs baseline: 2.1501x; 2.1501x over previous
import functools

import jax
import jax.numpy as jnp
from jax import lax
from jax.experimental import pallas as pl
from jax.experimental.pallas import tpu as pltpu
from jax.experimental.pallas import tpu_sc as plsc

Q = 1024
D = 32
N_TRAIN = 100000
N_NEIGH = 5
N_CLASSES = 100
KB = 2048
NKB = 50
KPAD = KB * NKB
RW = 8
BIG_IDX = 2**30


def _topk_body(qsq_ref, ksq_ref, xq_ref, xt_ref, out_ref, rval_ref, ridx_ref):
    k = pl.program_id(0)
    nk = pl.num_programs(0)

    @pl.when(k == 0)
    def _init():
        rval_ref[...] = jnp.full((Q, RW), jnp.inf, jnp.float32)
        ridx_ref[...] = jnp.full((Q, RW), BIG_IDX, jnp.int32)

    xq = xq_ref[...]
    xt = xt_ref[...]
    qsq = qsq_ref[...]
    ksq = ksq_ref[...]
    mm = jnp.dot(xq.astype(jnp.bfloat16), xt.astype(jnp.bfloat16),
                 preferred_element_type=jnp.float32)
    d2 = (qsq + ksq) - 2.0 * mm
    dist = jnp.sqrt(jnp.maximum(d2, 0.0))

    gidx = k * KB + lax.broadcasted_iota(jnp.int32, (Q, KB), 1)
    dist = jnp.where(gidx < N_TRAIN, dist, jnp.inf)

    vals = dist
    bvals, bidx = [], []
    for _ in range(N_NEIGH):
        m = jnp.min(vals, axis=1, keepdims=True)
        am = jnp.min(jnp.where(vals == m, gidx, BIG_IDX), axis=1,
                     keepdims=True)
        vals = jnp.where(gidx == am, jnp.inf, vals)
        bvals.append(m)
        bidx.append(am)

    cval = jnp.concatenate([rval_ref[...]] + bvals, axis=1)
    cidx = jnp.concatenate([ridx_ref[...]] + bidx, axis=1)
    nvals, nidx = [], []
    for _ in range(N_NEIGH):
        m = jnp.min(cval, axis=1, keepdims=True)
        am = jnp.min(jnp.where(cval == m, cidx, BIG_IDX), axis=1,
                     keepdims=True)
        cval = jnp.where(cidx == am, jnp.inf, cval)
        nvals.append(m)
        nidx.append(am)
    rval_ref[...] = jnp.concatenate(
        nvals + [jnp.full((Q, RW - N_NEIGH), jnp.inf, jnp.float32)], axis=1)
    ridx_ref[...] = jnp.concatenate(
        nidx + [jnp.full((Q, RW - N_NEIGH), BIG_IDX, jnp.int32)], axis=1)

    @pl.when(k == nk - 1)
    def _fin():
        li = lax.broadcasted_iota(jnp.int32, (Q, RW), 1)
        out_ref[...] = jnp.where(li < N_NEIGH, ridx_ref[...], 0)


def _tc_topk(qsq, ksq, x_test, xt):
    return pl.pallas_call(
        _topk_body,
        grid=(NKB,),
        in_specs=[
            pl.BlockSpec((Q, 1), lambda k: (0, 0)),
            pl.BlockSpec((1, KB), lambda k: (0, k)),
            pl.BlockSpec((Q, D), lambda k: (0, 0)),
            pl.BlockSpec((D, KB), lambda k: (0, k)),
        ],
        out_specs=pl.BlockSpec((Q, RW), lambda k: (0, 0)),
        out_shape=jax.ShapeDtypeStruct((Q, RW), jnp.int32),
        scratch_shapes=[
            pltpu.VMEM((Q, RW), jnp.float32),
            pltpu.VMEM((Q, RW), jnp.int32),
        ],
    )(qsq, ksq, x_test, xt)


W_OUT = Q // 32


def _sc_gather_mode_body(y_hbm, idx_hbm, out_hbm,
                         i0, i1, i2, i3, i4, l0, l1, l2, l3, l4,
                         out_v, sem):
    c = lax.axis_index("c")
    s = lax.axis_index("s")
    wid = s * 2 + c
    qbase = wid * W_OUT
    idx_bufs = [i0, i1, i2, i3, i4]
    lab_bufs = [l0, l1, l2, l3, l4]
    for j in range(N_NEIGH):
        pltpu.sync_copy(idx_hbm.at[pl.ds(j * Q + qbase, W_OUT)], idx_bufs[j])
    for j in range(N_NEIGH):
        pltpu.async_copy(y_hbm.at[idx_bufs[j]], lab_bufs[j], sem).wait()

    one = jnp.full((16,), 1, jnp.int32)
    zero = jnp.full((16,), 0, jnp.int32)
    for c2 in range(2):
        labs = [lab_bufs[j][pl.ds(c2 * 16, 16)] for j in range(N_NEIGH)]
        best = None
        pred = None
        for j in range(N_NEIGH):
            cnt = zero
            for j2 in range(N_NEIGH):
                cnt = cnt + jnp.where(labs[j] == labs[j2], one, zero)
            score = cnt * (N_CLASSES * 10) - labs[j]
            if j == 0:
                best, pred = score, labs[0]
            else:
                upd = score > best
                pred = jnp.where(upd, labs[j], pred)
                best = jnp.where(upd, score, best)
        out_v[pl.ds(c2 * 16, 16)] = pred
    pltpu.sync_copy(out_v, out_hbm.at[pl.ds(qbase, W_OUT)])


def _sc_gather_mode(y_train, idx_nm):
    mesh = plsc.VectorSubcoreMesh(core_axis_name="c", subcore_axis_name="s")
    f = pl.kernel(
        _sc_gather_mode_body,
        mesh=mesh,
        out_type=jax.ShapeDtypeStruct((Q,), jnp.int32),
        scratch_types=(
            [pltpu.VMEM((W_OUT,), jnp.int32) for _ in range(10)]
            + [pltpu.VMEM((W_OUT,), jnp.int32), pltpu.SemaphoreType.DMA]
        ),
    )
    return f(y_train, idx_nm)


def kernel(X_train, X_test, y_train):
    q_sq = jnp.sum(X_test * X_test, axis=-1, keepdims=True)
    k_sq = jnp.sum(X_train * X_train, axis=-1)
    xt = jnp.pad(X_train, ((0, KPAD - N_TRAIN), (0, 0))).T
    ksq = jnp.pad(k_sq, (0, KPAD - N_TRAIN))[None, :]
    idx = _tc_topk(q_sq, ksq, X_test, xt)
    idx_nm = idx[:, :N_NEIGH].T.reshape(-1)
    return _sc_gather_mode(y_train, idx_nm)


# speedup vs baseline: 2.8150x; 1.3092x over previous
import functools

import jax
import jax.numpy as jnp
from jax import lax
from jax.experimental import pallas as pl
from jax.experimental.pallas import tpu as pltpu
from jax.experimental.pallas import tpu_sc as plsc

Q = 1024
D = 32
N_TRAIN = 100000
N_NEIGH = 5
N_CLASSES = 100
KB = 2048
NKB = 50
KPAD = KB * NKB
RW = 8
BIG_IDX = 2**30


BIG_F = float(2**30)


def _topk_body(qsq_ref, ksq_ref, xq_ref, xt_ref, bv_ref, bi_ref):
    k = pl.program_id(0)

    mm = jnp.dot(xq_ref[...], xt_ref[...], preferred_element_type=jnp.float32)
    d2 = (qsq_ref[...] + ksq_ref[...]) - mm
    dist = jnp.sqrt(jnp.maximum(d2, 0.0))

    gidx = ((k * KB).astype(jnp.float32)
            + lax.broadcasted_iota(jnp.int32, (Q, KB), 1).astype(jnp.float32))

    vals = dist
    bvals, bidx = [], []
    for _ in range(N_NEIGH):
        m = jnp.min(vals, axis=1, keepdims=True)
        am = jnp.min(jnp.where(vals == m, gidx, BIG_F), axis=1,
                     keepdims=True)
        vals = jnp.where(gidx == am, jnp.inf, vals)
        bvals.append(m)
        bidx.append(am)
    bv_ref[0] = jnp.concatenate(
        bvals + [jnp.full((Q, RW - N_NEIGH), jnp.inf, jnp.float32)], axis=1)
    bi_ref[0] = jnp.concatenate(
        bidx + [jnp.full((Q, RW - N_NEIGH), BIG_F, jnp.float32)], axis=1)


def _merge_body(cv_ref, ci_ref, out_ref):
    cv = cv_ref[...]
    ci = ci_ref[...]
    outs = []
    for _ in range(N_NEIGH):
        m = jnp.min(cv, axis=1, keepdims=True)
        am = jnp.min(jnp.where(cv == m, ci, BIG_F), axis=1, keepdims=True)
        cv = jnp.where(ci == am, jnp.inf, cv)
        outs.append(am)
    oi = jnp.concatenate(
        outs + [jnp.zeros((Q, RW - N_NEIGH), jnp.float32)], axis=1)
    out_ref[...] = oi.astype(jnp.int32)


def _tc_topk(qsq, ksq, x_test, xt):
    bv, bi = pl.pallas_call(
        _topk_body,
        grid=(NKB,),
        in_specs=[
            pl.BlockSpec((Q, 1), lambda k: (0, 0)),
            pl.BlockSpec((1, KB), lambda k: (0, k)),
            pl.BlockSpec((Q, D), lambda k: (0, 0)),
            pl.BlockSpec((D, KB), lambda k: (0, k)),
        ],
        out_specs=[
            pl.BlockSpec((1, Q, RW), lambda k: (k, 0, 0)),
            pl.BlockSpec((1, Q, RW), lambda k: (k, 0, 0)),
        ],
        out_shape=[
            jax.ShapeDtypeStruct((NKB, Q, RW), jnp.float32),
            jax.ShapeDtypeStruct((NKB, Q, RW), jnp.float32),
        ],
    )(qsq, ksq, x_test, xt)
    cv = bv.transpose(1, 0, 2).reshape(Q, NKB * RW)
    ci = bi.transpose(1, 0, 2).reshape(Q, NKB * RW)
    return pl.pallas_call(
        _merge_body,
        out_shape=jax.ShapeDtypeStruct((Q, RW), jnp.int32),
    )(cv, ci)


W_OUT = Q // 32


def _sc_gather_mode_body(y_hbm, idx_hbm, out_hbm,
                         i0, i1, i2, i3, i4, l0, l1, l2, l3, l4,
                         out_v, sem):
    c = lax.axis_index("c")
    s = lax.axis_index("s")
    wid = s * 2 + c
    qbase = wid * W_OUT
    idx_bufs = [i0, i1, i2, i3, i4]
    lab_bufs = [l0, l1, l2, l3, l4]
    for j in range(N_NEIGH):
        pltpu.sync_copy(idx_hbm.at[pl.ds(j * Q + qbase, W_OUT)], idx_bufs[j])
    for j in range(N_NEIGH):
        pltpu.async_copy(y_hbm.at[idx_bufs[j]], lab_bufs[j], sem).wait()

    one = jnp.full((16,), 1, jnp.int32)
    zero = jnp.full((16,), 0, jnp.int32)
    for c2 in range(2):
        labs = [lab_bufs[j][pl.ds(c2 * 16, 16)] for j in range(N_NEIGH)]
        best = None
        pred = None
        for j in range(N_NEIGH):
            cnt = zero
            for j2 in range(N_NEIGH):
                cnt = cnt + jnp.where(labs[j] == labs[j2], one, zero)
            score = cnt * (N_CLASSES * 10) - labs[j]
            if j == 0:
                best, pred = score, labs[0]
            else:
                upd = score > best
                pred = jnp.where(upd, labs[j], pred)
                best = jnp.where(upd, score, best)
        out_v[pl.ds(c2 * 16, 16)] = pred
    pltpu.sync_copy(out_v, out_hbm.at[pl.ds(qbase, W_OUT)])


def _sc_gather_mode(y_train, idx_nm):
    mesh = plsc.VectorSubcoreMesh(core_axis_name="c", subcore_axis_name="s")
    f = pl.kernel(
        _sc_gather_mode_body,
        mesh=mesh,
        out_type=jax.ShapeDtypeStruct((Q,), jnp.int32),
        scratch_types=(
            [pltpu.VMEM((W_OUT,), jnp.int32) for _ in range(10)]
            + [pltpu.VMEM((W_OUT,), jnp.int32), pltpu.SemaphoreType.DMA]
        ),
    )
    return f(y_train, idx_nm)


def kernel(X_train, X_test, y_train):
    q_sq = jnp.sum(X_test * X_test, axis=-1, keepdims=True)
    k_sq = jnp.sum(X_train * X_train, axis=-1)
    xq2 = (X_test.astype(jnp.bfloat16) * jnp.bfloat16(2.0))
    xt = jnp.pad(X_train, ((0, KPAD - N_TRAIN), (0, 0))).T.astype(jnp.bfloat16)
    ksq = jnp.pad(k_sq, (0, KPAD - N_TRAIN),
                  constant_values=jnp.inf)[None, :]
    idx = _tc_topk(q_sq, ksq, xq2, xt)
    idx_nm = idx[:, :N_NEIGH].T.reshape(-1)
    return _sc_gather_mode(y_train, idx_nm)
